# baseline (speedup 1.0000x reference)
import functools

import jax
import jax.numpy as jnp
from jax import lax
from jax.experimental import pallas as pl
from jax.experimental.pallas import tpu as pltpu
from jax.experimental.pallas import tpu_sc as plsc

_HIDDEN = 768
_CHUNK = 32


@functools.lru_cache(maxsize=None)
def _make_gather(n_ids: int, vocab: int, hidden: int):
    info = plsc.get_sparse_core_info()
    nw = info.num_cores * info.num_subcores
    assert n_ids % (8 * nw) == 0
    per_w = n_ids // nw
    chunk = min(_CHUNK, per_w)
    n_chunks = per_w // chunk
    assert per_w % chunk == 0
    assert n_chunks % 4 == 0 and n_chunks >= 8

    mesh = plsc.VectorSubcoreMesh(core_axis_name="c", subcore_axis_name="s")

    buf_t = pltpu.VMEM((chunk, hidden), jnp.float32)

    @functools.partial(
        pl.kernel,
        mesh=mesh,
        out_type=jax.ShapeDtypeStruct((n_ids, hidden), jnp.float32),
        scratch_types=[
            pltpu.VMEM((per_w,), jnp.int32),
            buf_t, buf_t, buf_t, buf_t,
            pltpu.SemaphoreType.DMA, pltpu.SemaphoreType.DMA,
            pltpu.SemaphoreType.DMA, pltpu.SemaphoreType.DMA,
            pltpu.SemaphoreType.DMA, pltpu.SemaphoreType.DMA,
            pltpu.SemaphoreType.DMA, pltpu.SemaphoreType.DMA,
        ],
    )
    def gather_kernel(idx_hbm, table_hbm, out_hbm, idx_v,
                      b0, b1, b2, b3, sg0, sg1, sg2, sg3,
                      so0, so1, so2, so3):
        wid = lax.axis_index("s") * info.num_cores + lax.axis_index("c")
        base = wid * per_w
        pltpu.sync_copy(idx_hbm.at[pl.ds(base, per_w)], idx_v)

        bufs = (b0, b1, b2, b3)
        sgs = (sg0, sg1, sg2, sg3)
        sos = (so0, so1, so2, so3)

        def g_copy(j, c):
            return pltpu.make_async_copy(
                table_hbm.at[idx_v.at[pl.ds(c * chunk, chunk)]], bufs[j], sgs[j])

        def o_copy(j, c):
            return pltpu.make_async_copy(
                bufs[j], out_hbm.at[pl.ds(base + c * chunk, chunk)], sos[j])

        g_copy(0, 0).start()
        g_copy(1, 1).start()
        for j in range(4):
            g_copy(j, j).wait()
            o_copy(j, j).start()
            if j < 2:
                g_copy(j + 2, j + 2).start()
            else:
                o_copy(j - 2, j - 2).wait()
                g_copy(j - 2, j + 2).start()

        def body(k, carry):
            i0 = 4 * k
            for j in range(4):
                g_copy(j, i0 + j).wait()
                o_copy(j, i0 + j).start()
                j2 = (j + 2) % 4
                o_copy(j2, i0 + j - 2).wait()
                g_copy(j2, i0 + j + 2).start()
            return carry

        lax.fori_loop(1, n_chunks // 4 - 1, body, 0)

        i0 = n_chunks - 4
        for j in range(4):
            g_copy(j, i0 + j).wait()
            o_copy(j, i0 + j).start()
            if j < 2:
                j2 = j + 2
                o_copy(j2, i0 + j - 2).wait()
                g_copy(j2, i0 + j + 2).start()
        for j in range(4):
            o_copy(j, i0 + j).wait()

    return gather_kernel


def kernel(position_ids, table):
    batch, seq = position_ids.shape
    vocab, hidden = table.shape
    ids = position_ids.reshape(-1).astype(jnp.int32)
    out = _make_gather(ids.shape[0], vocab, hidden)(ids, table)
    return out.reshape(batch, seq, hidden)


# speedup vs baseline: 1.0113x; 1.0113x over previous
import functools

import jax
import jax.numpy as jnp
from jax import lax
from jax.experimental import pallas as pl
from jax.experimental.pallas import tpu as pltpu
from jax.experimental.pallas import tpu_sc as plsc

_CHUNK = 64


@functools.lru_cache(maxsize=None)
def _make_gather(n_ids: int, vocab: int, hidden: int):
    info = plsc.get_sparse_core_info()
    nw = info.num_cores * info.num_subcores
    assert n_ids % (8 * nw) == 0
    per_w = n_ids // nw
    chunk = min(_CHUNK, per_w)
    n_chunks = per_w // chunk
    assert per_w % chunk == 0
    assert n_chunks >= 4 and n_chunks % 2 == 0

    mesh = plsc.VectorSubcoreMesh(core_axis_name="c", subcore_axis_name="s")

    @functools.partial(
        pl.kernel,
        mesh=mesh,
        out_type=jax.ShapeDtypeStruct((n_ids, hidden), jnp.float32),
        scratch_types=[
            pltpu.VMEM((per_w,), jnp.int32),
            pltpu.VMEM((chunk, hidden), jnp.float32),
            pltpu.VMEM((chunk, hidden), jnp.float32),
            pltpu.SemaphoreType.DMA,
            pltpu.SemaphoreType.DMA,
            pltpu.SemaphoreType.DMA,
            pltpu.SemaphoreType.DMA,
        ],
    )
    def gather_kernel(idx_hbm, table_hbm, out_hbm, idx_v, buf0, buf1,
                      sg0, sg1, so0, so1):
        wid = lax.axis_index("s") * info.num_cores + lax.axis_index("c")
        base = wid * per_w
        pltpu.sync_copy(idx_hbm.at[pl.ds(base, per_w)], idx_v)

        def g_copy(buf, sem, c):
            return pltpu.make_async_copy(
                table_hbm.at[idx_v.at[pl.ds(c * chunk, chunk)]], buf, sem)

        def o_copy(buf, sem, c):
            return pltpu.make_async_copy(
                buf, out_hbm.at[pl.ds(base + c * chunk, chunk)], sem)

        g_copy(buf0, sg0, 0).start()
        g_copy(buf1, sg1, 1).start()
        g_copy(buf0, sg0, 0).wait()
        o_copy(buf0, so0, 0).start()

        def body(k, carry):
            o_copy(buf0, so0, 2 * k).wait()
            g_copy(buf0, sg0, 2 * k + 2).start()
            g_copy(buf1, sg1, 2 * k + 1).wait()
            o_copy(buf1, so1, 2 * k + 1).start()
            o_copy(buf1, so1, 2 * k + 1).wait()
            g_copy(buf1, sg1, 2 * k + 3).start()
            g_copy(buf0, sg0, 2 * k + 2).wait()
            o_copy(buf0, so0, 2 * k + 2).start()
            return carry

        lax.fori_loop(0, (n_chunks - 2) // 2, body, 0)

        g_copy(buf1, sg1, n_chunks - 1).wait()
        o_copy(buf0, so0, n_chunks - 2).wait()
        o_copy(buf1, so1, n_chunks - 1).start()
        o_copy(buf1, so1, n_chunks - 1).wait()

    return gather_kernel


def kernel(position_ids, table):
    batch, seq = position_ids.shape
    vocab, hidden = table.shape
    ids = position_ids.reshape(-1).astype(jnp.int32)
    out = _make_gather(ids.shape[0], vocab, hidden)(ids, table)
    return out.reshape(batch, seq, hidden)
